# baseline (speedup 1.0000x reference)
import jax
import jax.numpy as jnp
from jax import lax
from jax.experimental import pallas as pl
from jax.experimental.pallas import tpu as pltpu
from jax.experimental.pallas import tpu_sc as plsc

_N = 10000
_D = 512
_E = 10000
_R = 80
_NCHUNK = _N // _R
_G = 48
_U = 5
_LIST = _E + 2 * _G
_NLANE = 16
_NDSUB = _D // _NLANE


def _segmax_body(row_hbm, x_hbm, out_hbm, rows_v, elist, dlist, out_c,
                 stage_a, stage_b, sem_a, sem_b):
    nc = 2
    wid = lax.axis_index("s") * nc + lax.axis_index("c")
    nw_total = nc * 16

    pltpu.sync_copy(row_hbm, rows_v)

    def do_chunk(k, _):
        chunk = k * nw_total + wid
        lo = pl.multiple_of(chunk * _R, 8)

        def init_row(r, c):
            for d in range(_NDSUB):
                out_c[r, pl.ds(d * _NLANE, _NLANE)] = jnp.full(
                    (_NLANE,), -jnp.inf, jnp.float32)
            return c
        lax.fori_loop(0, _R, init_row, 0)

        def scan80(i, count):
            base = i * (_U * _NLANE)
            vs, ms, pcs = [], [], []
            for u in range(_U):
                v = rows_v[pl.ds(base + u * _NLANE, _NLANE)]
                m = (v >= lo) & (v < lo + _R)
                vs.append(v)
                ms.append(m)
                pcs.append(plsc.all_reduce_population_count(m)[0])
            starts = []
            acc = count
            for u in range(_U):
                starts.append(acc)
                acc = acc + pcs[u]
            for u in range(_U):
                eids = lax.iota(jnp.int32, _NLANE) + (base + u * _NLANE)
                dest = starts[u] + plsc.cumsum(ms[u].astype(jnp.int32)) - 1
                plsc.store_scatter(elist, [dest], eids, mask=ms[u])
                plsc.store_scatter(dlist, [dest], vs[u] - lo, mask=ms[u])
            return acc
        count = lax.fori_loop(0, _E // (_U * _NLANE), scan80, jnp.int32(0))

        for t in range(_G // _NLANE):
            elist[pl.ds(count + t * _NLANE, _NLANE)] = jnp.zeros(
                (_NLANE,), jnp.int32)

        nwaves = (count + _G - 1) // _G

        def issue(w, stage, sem):
            pltpu.make_async_copy(x_hbm.at[elist.at[pl.ds(w * _G, _G)]],
                                  stage, sem).start()

        def process(w, stage, sem):
            g0 = w * _G
            pltpu.make_async_copy(x_hbm.at[elist.at[pl.ds(g0, _G)]], stage,
                                  sem).wait()
            gn = jnp.minimum(count - g0, _G)

            def edge(g, _c):
                l = dlist[pl.ds(g0 + g, _NLANE)][0]
                for d in range(_NDSUB):
                    sl = pl.ds(d * _NLANE, _NLANE)
                    out_c[l, sl] = jnp.maximum(out_c[l, sl], stage[g, sl])
                return _c
            lax.fori_loop(0, gn, edge, 0)

        @pl.when(nwaves > 0)
        def _():
            issue(0, stage_a, sem_a)

        def wave(w, _):
            even = (w % 2) == 0

            @pl.when(even)
            def _():
                @pl.when(w + 1 < nwaves)
                def _():
                    issue(w + 1, stage_b, sem_b)
                process(w, stage_a, sem_a)

            @pl.when(jnp.logical_not(even))
            def _():
                @pl.when(w + 1 < nwaves)
                def _():
                    issue(w + 1, stage_a, sem_a)
                process(w, stage_b, sem_b)
            return 0
        lax.fori_loop(0, nwaves, wave, 0)

        pltpu.sync_copy(out_c, out_hbm.at[pl.ds(lo, _R)])
        return 0

    nk = (_NCHUNK - wid + nw_total - 1) // nw_total
    lax.fori_loop(0, nk, do_chunk, 0)


def _segmax(row, x):
    mesh = plsc.VectorSubcoreMesh(core_axis_name="c", subcore_axis_name="s")
    return pl.kernel(
        _segmax_body,
        out_type=jax.ShapeDtypeStruct((_N, _D), jnp.float32),
        mesh=mesh,
        compiler_params=pltpu.CompilerParams(needs_layout_passes=False),
        scratch_types=[
            pltpu.VMEM((_E,), jnp.int32),
            pltpu.VMEM((_LIST,), jnp.int32),
            pltpu.VMEM((_LIST,), jnp.int32),
            pltpu.VMEM((_R, _D), jnp.float32),
            pltpu.VMEM((_G, _D), jnp.float32),
            pltpu.VMEM((_G, _D), jnp.float32),
            pltpu.SemaphoreType.DMA,
            pltpu.SemaphoreType.DMA,
        ],
    )(row, x)


_BM = 1000


def _mm_body(seg_ref, x_ref, w_ref, b_ref, o_ref):
    seg = seg_ref[...]
    agg = jnp.where(seg == -jnp.inf, x_ref[...], seg)
    acc = lax.dot_general(agg, w_ref[...], (((1,), (1,)), ((), ())),
                          preferred_element_type=jnp.float32)
    o_ref[...] = acc + b_ref[...]


def _matmul(seg, x, W, b2d):
    return pl.pallas_call(
        _mm_body,
        grid=(_N // _BM,),
        in_specs=[
            pl.BlockSpec((_BM, _D), lambda i: (i, 0)),
            pl.BlockSpec((_BM, _D), lambda i: (i, 0)),
            pl.BlockSpec((_D, _D), lambda i: (0, 0)),
            pl.BlockSpec((1, _D), lambda i: (0, 0)),
        ],
        out_specs=pl.BlockSpec((_BM, _D), lambda i: (i, 0)),
        out_shape=jax.ShapeDtypeStruct((_N, _D), jnp.float32),
    )(seg, x, W, b2d)


@jax.jit
def kernel(x, edge_index, W, b):
    row = edge_index[0]
    seg = _segmax(row, x)
    return _matmul(seg, x, W, b.reshape(1, _D))


# speedup vs baseline: 1.3554x; 1.3554x over previous
import jax
import jax.numpy as jnp
from jax import lax
from jax.experimental import pallas as pl
from jax.experimental.pallas import tpu as pltpu
from jax.experimental.pallas import tpu_sc as plsc

_N = 10000
_D = 512
_E = 10000
_R = 80
_NCHUNK = _N // _R
_G = 16
_U = 5
_LIST = _E + 2 * _G
_NLANE = 16
_NDSUB = _D // _NLANE


def _segmax_body(row_hbm, x_hbm, out_hbm, rows_v, elist, dlist, out_c,
                 stage_a, stage_b, sem_a, sem_b):
    nc = 2
    wid = lax.axis_index("s") * nc + lax.axis_index("c")
    nw_total = nc * 16

    pltpu.sync_copy(row_hbm, rows_v)

    def do_chunk(k, _):
        chunk = k * nw_total + wid
        lo = pl.multiple_of(chunk * _R, 8)

        def init_row(r, c):
            for d in range(_NDSUB):
                out_c[r, pl.ds(d * _NLANE, _NLANE)] = jnp.full(
                    (_NLANE,), -jnp.inf, jnp.float32)
            return c
        lax.fori_loop(0, _R, init_row, 0)

        def scan80(i, count):
            base = i * (_U * _NLANE)
            vs, ms, pcs = [], [], []
            for u in range(_U):
                v = rows_v[pl.ds(base + u * _NLANE, _NLANE)]
                m = (v >= lo) & (v < lo + _R)
                vs.append(v)
                ms.append(m)
                pcs.append(plsc.all_reduce_population_count(m)[0])
            starts = []
            acc = count
            for u in range(_U):
                starts.append(acc)
                acc = acc + pcs[u]
            for u in range(_U):
                eids = lax.iota(jnp.int32, _NLANE) + (base + u * _NLANE)
                dest = starts[u] + plsc.cumsum(ms[u].astype(jnp.int32)) - 1
                plsc.store_scatter(elist, [dest], eids, mask=ms[u])
                plsc.store_scatter(dlist, [dest], vs[u] - lo, mask=ms[u])
            return acc
        count = lax.fori_loop(0, _E // (_U * _NLANE), scan80, jnp.int32(0))

        for t in range(_G // _NLANE):
            elist[pl.ds(count + t * _NLANE, _NLANE)] = jnp.zeros(
                (_NLANE,), jnp.int32)

        nwaves = (count + _G - 1) // _G

        def issue(w, stage, sem):
            pltpu.make_async_copy(x_hbm.at[elist.at[pl.ds(w * _G, _G)]],
                                  stage, sem).start()

        def process(w, stage, sem):
            g0 = w * _G
            pltpu.make_async_copy(x_hbm.at[elist.at[pl.ds(g0, _G)]], stage,
                                  sem).wait()
            gn = jnp.minimum(count - g0, _G)

            def edge(g, _c):
                l = dlist[pl.ds(g0 + g, _NLANE)][0]
                for d in range(_NDSUB):
                    sl = pl.ds(d * _NLANE, _NLANE)
                    out_c[l, sl] = jnp.maximum(out_c[l, sl], stage[g, sl])
                return _c
            lax.fori_loop(0, gn, edge, 0)

        @pl.when(nwaves > 0)
        def _():
            issue(0, stage_a, sem_a)

        def wave(w, _):
            even = (w % 2) == 0

            @pl.when(even)
            def _():
                @pl.when(w + 1 < nwaves)
                def _():
                    issue(w + 1, stage_b, sem_b)
                process(w, stage_a, sem_a)

            @pl.when(jnp.logical_not(even))
            def _():
                @pl.when(w + 1 < nwaves)
                def _():
                    issue(w + 1, stage_a, sem_a)
                process(w, stage_b, sem_b)
            return 0
        lax.fori_loop(0, nwaves, wave, 0)

        pltpu.sync_copy(out_c, out_hbm.at[pl.ds(lo, _R)])
        return 0

    nk = (_NCHUNK - wid + nw_total - 1) // nw_total
    lax.fori_loop(0, nk, do_chunk, 0)


def _segmax(row, x):
    mesh = plsc.VectorSubcoreMesh(core_axis_name="c", subcore_axis_name="s")
    return pl.kernel(
        _segmax_body,
        out_type=jax.ShapeDtypeStruct((_N, _D), jnp.float32),
        mesh=mesh,
        compiler_params=pltpu.CompilerParams(needs_layout_passes=False),
        scratch_types=[
            pltpu.VMEM((_E,), jnp.int32),
            pltpu.VMEM((_LIST,), jnp.int32),
            pltpu.VMEM((_LIST,), jnp.int32),
            pltpu.VMEM((_R, _D), jnp.float32),
            pltpu.VMEM((_G, _D), jnp.float32),
            pltpu.VMEM((_G, _D), jnp.float32),
            pltpu.SemaphoreType.DMA,
            pltpu.SemaphoreType.DMA,
        ],
    )(row, x)


_BM = 1000


def _mm_body(seg_ref, x_ref, w_ref, b_ref, o_ref):
    seg = seg_ref[...]
    agg = jnp.where(seg == -jnp.inf, x_ref[...], seg)
    acc = lax.dot_general(agg, w_ref[...], (((1,), (1,)), ((), ())),
                          preferred_element_type=jnp.float32)
    o_ref[...] = acc + b_ref[...]


def _matmul(seg, x, W, b2d):
    return pl.pallas_call(
        _mm_body,
        grid=(_N // _BM,),
        in_specs=[
            pl.BlockSpec((_BM, _D), lambda i: (i, 0)),
            pl.BlockSpec((_BM, _D), lambda i: (i, 0)),
            pl.BlockSpec((_D, _D), lambda i: (0, 0)),
            pl.BlockSpec((1, _D), lambda i: (0, 0)),
        ],
        out_specs=pl.BlockSpec((_BM, _D), lambda i: (i, 0)),
        out_shape=jax.ShapeDtypeStruct((_N, _D), jnp.float32),
    )(seg, x, W, b2d)


@jax.jit
def kernel(x, edge_index, W, b):
    row = edge_index[0]
    seg = _segmax(row, x)
    return _matmul(seg, x, W, b.reshape(1, _D))


# speedup vs baseline: 1.4103x; 1.0405x over previous
import jax
import jax.numpy as jnp
from jax import lax
from jax.experimental import pallas as pl
from jax.experimental.pallas import tpu as pltpu
from jax.experimental.pallas import tpu_sc as plsc

_N = 10000
_D = 512
_E = 10000
_R = 80
_NCHUNK = _N // _R
_G = 8
_U = 5
_LIST = _E + 2 * _G
_NLANE = 16
_NDSUB = _D // _NLANE


def _segmax_body(row_hbm, x_hbm, out_hbm, rows_v, elist, dlist, out_c,
                 stage_a, stage_b, sem_a, sem_b):
    nc = 2
    wid = lax.axis_index("s") * nc + lax.axis_index("c")
    nw_total = nc * 16

    pltpu.sync_copy(row_hbm, rows_v)

    def do_chunk(k, _):
        chunk = k * nw_total + wid
        lo = pl.multiple_of(chunk * _R, 8)

        def init_row(r, c):
            for d in range(_NDSUB):
                out_c[r, pl.ds(d * _NLANE, _NLANE)] = jnp.full(
                    (_NLANE,), -jnp.inf, jnp.float32)
            return c
        lax.fori_loop(0, _R, init_row, 0)

        def scan80(i, count):
            base = i * (_U * _NLANE)
            vs, ms, pcs = [], [], []
            for u in range(_U):
                v = rows_v[pl.ds(base + u * _NLANE, _NLANE)]
                m = (v >= lo) & (v < lo + _R)
                vs.append(v)
                ms.append(m)
                pcs.append(plsc.all_reduce_population_count(m)[0])
            starts = []
            acc = count
            for u in range(_U):
                starts.append(acc)
                acc = acc + pcs[u]
            for u in range(_U):
                eids = lax.iota(jnp.int32, _NLANE) + (base + u * _NLANE)
                dest = starts[u] + plsc.cumsum(ms[u].astype(jnp.int32)) - 1
                plsc.store_scatter(elist, [dest], eids, mask=ms[u])
                plsc.store_scatter(dlist, [dest], vs[u] - lo, mask=ms[u])
            return acc
        count = lax.fori_loop(0, _E // (_U * _NLANE), scan80, jnp.int32(0))

        for t in range(max(1, _G // _NLANE)):
            elist[pl.ds(count + t * _NLANE, _NLANE)] = jnp.zeros(
                (_NLANE,), jnp.int32)

        nwaves = (count + _G - 1) // _G

        def issue(w, stage, sem):
            pltpu.make_async_copy(x_hbm.at[elist.at[pl.ds(w * _G, _G)]],
                                  stage, sem).start()

        def process(w, stage, sem):
            g0 = w * _G
            pltpu.make_async_copy(x_hbm.at[elist.at[pl.ds(g0, _G)]], stage,
                                  sem).wait()
            gn = jnp.minimum(count - g0, _G)

            def edge(g, _c):
                l = dlist[pl.ds(g0 + g, _NLANE)][0]
                for d in range(_NDSUB):
                    sl = pl.ds(d * _NLANE, _NLANE)
                    out_c[l, sl] = jnp.maximum(out_c[l, sl], stage[g, sl])
                return _c
            lax.fori_loop(0, gn, edge, 0)

        @pl.when(nwaves > 0)
        def _():
            issue(0, stage_a, sem_a)

        def wave(w, _):
            even = (w % 2) == 0

            @pl.when(even)
            def _():
                @pl.when(w + 1 < nwaves)
                def _():
                    issue(w + 1, stage_b, sem_b)
                process(w, stage_a, sem_a)

            @pl.when(jnp.logical_not(even))
            def _():
                @pl.when(w + 1 < nwaves)
                def _():
                    issue(w + 1, stage_a, sem_a)
                process(w, stage_b, sem_b)
            return 0
        lax.fori_loop(0, nwaves, wave, 0)

        pltpu.sync_copy(out_c, out_hbm.at[pl.ds(lo, _R)])
        return 0

    nk = (_NCHUNK - wid + nw_total - 1) // nw_total
    lax.fori_loop(0, nk, do_chunk, 0)


def _segmax(row, x):
    mesh = plsc.VectorSubcoreMesh(core_axis_name="c", subcore_axis_name="s")
    return pl.kernel(
        _segmax_body,
        out_type=jax.ShapeDtypeStruct((_N, _D), jnp.float32),
        mesh=mesh,
        compiler_params=pltpu.CompilerParams(needs_layout_passes=False),
        scratch_types=[
            pltpu.VMEM((_E,), jnp.int32),
            pltpu.VMEM((_LIST,), jnp.int32),
            pltpu.VMEM((_LIST,), jnp.int32),
            pltpu.VMEM((_R, _D), jnp.float32),
            pltpu.VMEM((_G, _D), jnp.float32),
            pltpu.VMEM((_G, _D), jnp.float32),
            pltpu.SemaphoreType.DMA,
            pltpu.SemaphoreType.DMA,
        ],
    )(row, x)


_BM = 1000


def _mm_body(seg_ref, x_ref, w_ref, b_ref, o_ref):
    seg = seg_ref[...]
    agg = jnp.where(seg == -jnp.inf, x_ref[...], seg)
    acc = lax.dot_general(agg, w_ref[...], (((1,), (1,)), ((), ())),
                          preferred_element_type=jnp.float32)
    o_ref[...] = acc + b_ref[...]


def _matmul(seg, x, W, b2d):
    return pl.pallas_call(
        _mm_body,
        grid=(_N // _BM,),
        in_specs=[
            pl.BlockSpec((_BM, _D), lambda i: (i, 0)),
            pl.BlockSpec((_BM, _D), lambda i: (i, 0)),
            pl.BlockSpec((_D, _D), lambda i: (0, 0)),
            pl.BlockSpec((1, _D), lambda i: (0, 0)),
        ],
        out_specs=pl.BlockSpec((_BM, _D), lambda i: (i, 0)),
        out_shape=jax.ShapeDtypeStruct((_N, _D), jnp.float32),
    )(seg, x, W, b2d)


@jax.jit
def kernel(x, edge_index, W, b):
    row = edge_index[0]
    seg = _segmax(row, x)
    return _matmul(seg, x, W, b.reshape(1, _D))


# speedup vs baseline: 1.4287x; 1.0131x over previous
import jax
import jax.numpy as jnp
from jax import lax
from jax.experimental import pallas as pl
from jax.experimental.pallas import tpu as pltpu
from jax.experimental.pallas import tpu_sc as plsc

_N = 10000
_D = 512
_E = 10000
_R = 80
_NCHUNK = _N // _R
_G = 8
_U = 5
_LIST = _E + 2 * _G
_NLANE = 16
_NDSUB = _D // _NLANE


def _segmax_body(row_hbm, x_hbm, out_hbm, rows_v, elist, dlist, out_c,
                 stage_a, stage_b, stage_c, sem_a, sem_b, sem_c):
    nc = 2
    wid = lax.axis_index("s") * nc + lax.axis_index("c")
    nw_total = nc * 16

    pltpu.sync_copy(row_hbm, rows_v)

    def do_chunk(k, _):
        chunk = k * nw_total + wid
        lo = pl.multiple_of(chunk * _R, 8)

        def init_row(r, c):
            for d in range(_NDSUB):
                out_c[r, pl.ds(d * _NLANE, _NLANE)] = jnp.full(
                    (_NLANE,), -jnp.inf, jnp.float32)
            return c
        lax.fori_loop(0, _R, init_row, 0)

        def scan80(i, count):
            base = i * (_U * _NLANE)
            vs, ms, pcs = [], [], []
            for u in range(_U):
                v = rows_v[pl.ds(base + u * _NLANE, _NLANE)]
                m = (v >= lo) & (v < lo + _R)
                vs.append(v)
                ms.append(m)
                pcs.append(plsc.all_reduce_population_count(m)[0])
            starts = []
            acc = count
            for u in range(_U):
                starts.append(acc)
                acc = acc + pcs[u]
            for u in range(_U):
                eids = lax.iota(jnp.int32, _NLANE) + (base + u * _NLANE)
                dest = starts[u] + plsc.cumsum(ms[u].astype(jnp.int32)) - 1
                plsc.store_scatter(elist, [dest], eids, mask=ms[u])
                plsc.store_scatter(dlist, [dest], vs[u] - lo, mask=ms[u])
            return acc
        count = lax.fori_loop(0, _E // (_U * _NLANE), scan80, jnp.int32(0))

        for t in range(max(1, _G // _NLANE)):
            elist[pl.ds(count + t * _NLANE, _NLANE)] = jnp.zeros(
                (_NLANE,), jnp.int32)

        nwaves = (count + _G - 1) // _G

        def issue(w, stage, sem):
            pltpu.make_async_copy(x_hbm.at[elist.at[pl.ds(w * _G, _G)]],
                                  stage, sem).start()

        def process(w, stage, sem):
            g0 = w * _G
            pltpu.make_async_copy(x_hbm.at[elist.at[pl.ds(g0, _G)]], stage,
                                  sem).wait()
            gn = jnp.minimum(count - g0, _G)

            def edge(g, _c):
                l = dlist[pl.ds(g0 + g, _NLANE)][0]
                for d in range(_NDSUB):
                    sl = pl.ds(d * _NLANE, _NLANE)
                    out_c[l, sl] = jnp.maximum(out_c[l, sl], stage[g, sl])
                return _c
            lax.fori_loop(0, gn, edge, 0)

        bufs = ((stage_a, sem_a), (stage_b, sem_b), (stage_c, sem_c))

        @pl.when(nwaves > 0)
        def _():
            issue(0, stage_a, sem_a)

        @pl.when(nwaves > 1)
        def _():
            issue(1, stage_b, sem_b)

        def wave(w, _):
            r = w % 3
            for ri in range(3):
                @pl.when(r == ri)
                def _(ri=ri):
                    @pl.when(w + 2 < nwaves)
                    def _():
                        st, se = bufs[(ri + 2) % 3]
                        issue(w + 2, st, se)
                    st, se = bufs[ri]
                    process(w, st, se)
            return 0
        lax.fori_loop(0, nwaves, wave, 0)

        pltpu.sync_copy(out_c, out_hbm.at[pl.ds(lo, _R)])
        return 0

    nk = (_NCHUNK - wid + nw_total - 1) // nw_total
    lax.fori_loop(0, nk, do_chunk, 0)


def _segmax(row, x):
    mesh = plsc.VectorSubcoreMesh(core_axis_name="c", subcore_axis_name="s")
    return pl.kernel(
        _segmax_body,
        out_type=jax.ShapeDtypeStruct((_N, _D), jnp.float32),
        mesh=mesh,
        compiler_params=pltpu.CompilerParams(needs_layout_passes=False),
        scratch_types=[
            pltpu.VMEM((_E,), jnp.int32),
            pltpu.VMEM((_LIST,), jnp.int32),
            pltpu.VMEM((_LIST,), jnp.int32),
            pltpu.VMEM((_R, _D), jnp.float32),
            pltpu.VMEM((_G, _D), jnp.float32),
            pltpu.VMEM((_G, _D), jnp.float32),
            pltpu.VMEM((_G, _D), jnp.float32),
            pltpu.SemaphoreType.DMA,
            pltpu.SemaphoreType.DMA,
            pltpu.SemaphoreType.DMA,
        ],
    )(row, x)


_BM = 1000


def _mm_body(seg_ref, x_ref, w_ref, b_ref, o_ref):
    seg = seg_ref[...]
    agg = jnp.where(seg == -jnp.inf, x_ref[...], seg)
    acc = lax.dot_general(agg, w_ref[...], (((1,), (1,)), ((), ())),
                          preferred_element_type=jnp.float32)
    o_ref[...] = acc + b_ref[...]


def _matmul(seg, x, W, b2d):
    return pl.pallas_call(
        _mm_body,
        grid=(_N // _BM,),
        in_specs=[
            pl.BlockSpec((_BM, _D), lambda i: (i, 0)),
            pl.BlockSpec((_BM, _D), lambda i: (i, 0)),
            pl.BlockSpec((_D, _D), lambda i: (0, 0)),
            pl.BlockSpec((1, _D), lambda i: (0, 0)),
        ],
        out_specs=pl.BlockSpec((_BM, _D), lambda i: (i, 0)),
        out_shape=jax.ShapeDtypeStruct((_N, _D), jnp.float32),
    )(seg, x, W, b2d)


@jax.jit
def kernel(x, edge_index, W, b):
    row = edge_index[0]
    seg = _segmax(row, x)
    return _matmul(seg, x, W, b.reshape(1, _D))


# speedup vs baseline: 1.4462x; 1.0122x over previous
import jax
import jax.numpy as jnp
from jax import lax
from jax.experimental import pallas as pl
from jax.experimental.pallas import tpu as pltpu
from jax.experimental.pallas import tpu_sc as plsc

_N = 10000
_D = 512
_E = 10000
_R = 80
_NCHUNK = _N // _R
_G = 8
_U = 5
_LIST = _E + 2 * _G
_NLANE = 16
_NDSUB = _D // _NLANE


def _segmax_body(row_hbm, x_hbm, out_hbm, rows_v, elist, dlist, out_c,
                 stage_a, stage_b, stage_c, stage_d, sem_a, sem_b, sem_c,
                 sem_d):
    nc = 2
    wid = lax.axis_index("s") * nc + lax.axis_index("c")
    nw_total = nc * 16

    pltpu.sync_copy(row_hbm, rows_v)

    def do_chunk(k, _):
        chunk = k * nw_total + wid
        lo = pl.multiple_of(chunk * _R, 8)

        def init_row(r, c):
            for d in range(_NDSUB):
                out_c[r, pl.ds(d * _NLANE, _NLANE)] = jnp.full(
                    (_NLANE,), -jnp.inf, jnp.float32)
            return c
        lax.fori_loop(0, _R, init_row, 0)

        def scan80(i, count):
            base = i * (_U * _NLANE)
            vs, ms, pcs = [], [], []
            for u in range(_U):
                v = rows_v[pl.ds(base + u * _NLANE, _NLANE)]
                m = (v >= lo) & (v < lo + _R)
                vs.append(v)
                ms.append(m)
                pcs.append(plsc.all_reduce_population_count(m)[0])
            starts = []
            acc = count
            for u in range(_U):
                starts.append(acc)
                acc = acc + pcs[u]
            for u in range(_U):
                eids = lax.iota(jnp.int32, _NLANE) + (base + u * _NLANE)
                dest = starts[u] + plsc.cumsum(ms[u].astype(jnp.int32)) - 1
                plsc.store_scatter(elist, [dest], eids, mask=ms[u])
                plsc.store_scatter(dlist, [dest], vs[u] - lo, mask=ms[u])
            return acc
        count = lax.fori_loop(0, _E // (_U * _NLANE), scan80, jnp.int32(0))

        for t in range(max(1, _G // _NLANE)):
            elist[pl.ds(count + t * _NLANE, _NLANE)] = jnp.zeros(
                (_NLANE,), jnp.int32)

        nwaves = (count + _G - 1) // _G

        def issue(w, stage, sem):
            pltpu.make_async_copy(x_hbm.at[elist.at[pl.ds(w * _G, _G)]],
                                  stage, sem).start()

        def process(w, stage, sem):
            g0 = w * _G
            pltpu.make_async_copy(x_hbm.at[elist.at[pl.ds(g0, _G)]], stage,
                                  sem).wait()
            gn = jnp.minimum(count - g0, _G)

            def edge(g, _c):
                l = dlist[pl.ds(g0 + g, _NLANE)][0]
                for d in range(_NDSUB):
                    sl = pl.ds(d * _NLANE, _NLANE)
                    out_c[l, sl] = jnp.maximum(out_c[l, sl], stage[g, sl])
                return _c
            lax.fori_loop(0, gn, edge, 0)

        bufs = ((stage_a, sem_a), (stage_b, sem_b), (stage_c, sem_c),
                (stage_d, sem_d))

        for pi in range(3):
            @pl.when(nwaves > pi)
            def _(pi=pi):
                issue(pi, bufs[pi][0], bufs[pi][1])

        def wave(w, _):
            r = w % 4
            for ri in range(4):
                @pl.when(r == ri)
                def _(ri=ri):
                    @pl.when(w + 3 < nwaves)
                    def _():
                        st, se = bufs[(ri + 3) % 4]
                        issue(w + 3, st, se)
                    st, se = bufs[ri]
                    process(w, st, se)
            return 0
        lax.fori_loop(0, nwaves, wave, 0)

        pltpu.sync_copy(out_c, out_hbm.at[pl.ds(lo, _R)])
        return 0

    nk = (_NCHUNK - wid + nw_total - 1) // nw_total
    lax.fori_loop(0, nk, do_chunk, 0)


def _segmax(row, x):
    mesh = plsc.VectorSubcoreMesh(core_axis_name="c", subcore_axis_name="s")
    return pl.kernel(
        _segmax_body,
        out_type=jax.ShapeDtypeStruct((_N, _D), jnp.float32),
        mesh=mesh,
        compiler_params=pltpu.CompilerParams(needs_layout_passes=False),
        scratch_types=[
            pltpu.VMEM((_E,), jnp.int32),
            pltpu.VMEM((_LIST,), jnp.int32),
            pltpu.VMEM((_LIST,), jnp.int32),
            pltpu.VMEM((_R, _D), jnp.float32),
            pltpu.VMEM((_G, _D), jnp.float32),
            pltpu.VMEM((_G, _D), jnp.float32),
            pltpu.VMEM((_G, _D), jnp.float32),
            pltpu.VMEM((_G, _D), jnp.float32),
            pltpu.SemaphoreType.DMA,
            pltpu.SemaphoreType.DMA,
            pltpu.SemaphoreType.DMA,
            pltpu.SemaphoreType.DMA,
        ],
    )(row, x)


_BM = 1000


def _mm_body(seg_ref, x_ref, w_ref, b_ref, o_ref):
    seg = seg_ref[...]
    agg = jnp.where(seg == -jnp.inf, x_ref[...], seg)
    acc = lax.dot_general(agg, w_ref[...], (((1,), (1,)), ((), ())),
                          preferred_element_type=jnp.float32)
    o_ref[...] = acc + b_ref[...]


def _matmul(seg, x, W, b2d):
    return pl.pallas_call(
        _mm_body,
        grid=(_N // _BM,),
        in_specs=[
            pl.BlockSpec((_BM, _D), lambda i: (i, 0)),
            pl.BlockSpec((_BM, _D), lambda i: (i, 0)),
            pl.BlockSpec((_D, _D), lambda i: (0, 0)),
            pl.BlockSpec((1, _D), lambda i: (0, 0)),
        ],
        out_specs=pl.BlockSpec((_BM, _D), lambda i: (i, 0)),
        out_shape=jax.ShapeDtypeStruct((_N, _D), jnp.float32),
    )(seg, x, W, b2d)


@jax.jit
def kernel(x, edge_index, W, b):
    row = edge_index[0]
    seg = _segmax(row, x)
    return _matmul(seg, x, W, b.reshape(1, _D))


# speedup vs baseline: 1.4917x; 1.0315x over previous
import jax
import jax.numpy as jnp
from jax import lax
from jax.experimental import pallas as pl
from jax.experimental.pallas import tpu as pltpu
from jax.experimental.pallas import tpu_sc as plsc

_N = 10000
_D = 512
_E = 10000
_R = 80
_NCHUNK = _N // _R
_G = 8
_U = 5
_LIST = _E + 2 * _G
_NLANE = 16
_NDSUB = _D // _NLANE


def _segmax_body(row_hbm, x_hbm, out_hbm, rows_v, elist, dlist, out_c,
                 stage_a, stage_b, stage_c, stage_d, sem_a, sem_b, sem_c,
                 sem_d):
    nc = 2
    wid = lax.axis_index("s") * nc + lax.axis_index("c")
    nw_total = nc * 16

    pltpu.sync_copy(row_hbm, rows_v)

    def do_chunk(k, _):
        chunk = k * nw_total + wid
        lo = pl.multiple_of(chunk * _R, 8)

        def scan80(i, count):
            base = i * (_U * _NLANE)
            vs, ms, pcs = [], [], []
            for u in range(_U):
                v = rows_v[pl.ds(base + u * _NLANE, _NLANE)]
                m = (v >= lo) & (v < lo + _R)
                vs.append(v)
                ms.append(m)
                pcs.append(plsc.all_reduce_population_count(m)[0])
            starts = []
            acc = count
            for u in range(_U):
                starts.append(acc)
                acc = acc + pcs[u]
            for u in range(_U):
                eids = lax.iota(jnp.int32, _NLANE) + (base + u * _NLANE)
                dest = starts[u] + plsc.cumsum(ms[u].astype(jnp.int32)) - 1
                plsc.store_scatter(elist, [dest], eids, mask=ms[u])
                plsc.store_scatter(dlist, [dest], vs[u] - lo, mask=ms[u])
            return acc
        count = lax.fori_loop(0, _E // (_U * _NLANE), scan80, jnp.int32(0))

        for t in range(max(1, _G // _NLANE)):
            elist[pl.ds(count + t * _NLANE, _NLANE)] = jnp.zeros(
                (_NLANE,), jnp.int32)

        nwaves = (count + _G - 1) // _G

        def issue(w, stage, sem):
            pltpu.make_async_copy(x_hbm.at[elist.at[pl.ds(w * _G, _G)]],
                                  stage, sem).start()

        def process(w, stage, sem):
            g0 = w * _G
            pltpu.make_async_copy(x_hbm.at[elist.at[pl.ds(g0, _G)]], stage,
                                  sem).wait()
            gn = jnp.minimum(count - g0, _G)

            def edge(g, _c):
                l = dlist[pl.ds(g0 + g, _NLANE)][0]
                for d in range(_NDSUB):
                    sl = pl.ds(d * _NLANE, _NLANE)
                    out_c[l, sl] = jnp.maximum(out_c[l, sl], stage[g, sl])
                return _c
            lax.fori_loop(0, gn, edge, 0)

        bufs = ((stage_a, sem_a), (stage_b, sem_b), (stage_c, sem_c),
                (stage_d, sem_d))

        for pi in range(3):
            @pl.when(nwaves > pi)
            def _(pi=pi):
                issue(pi, bufs[pi][0], bufs[pi][1])

        def init_row(r, c):
            for d in range(_NDSUB):
                out_c[r, pl.ds(d * _NLANE, _NLANE)] = jnp.full(
                    (_NLANE,), -jnp.inf, jnp.float32)
            return c
        lax.fori_loop(0, _R, init_row, 0)


        def wave(w, _):
            r = w % 4
            for ri in range(4):
                @pl.when(r == ri)
                def _(ri=ri):
                    @pl.when(w + 3 < nwaves)
                    def _():
                        st, se = bufs[(ri + 3) % 4]
                        issue(w + 3, st, se)
                    st, se = bufs[ri]
                    process(w, st, se)
            return 0
        lax.fori_loop(0, nwaves, wave, 0)

        pltpu.sync_copy(out_c, out_hbm.at[pl.ds(lo, _R)])
        return 0

    nk = (_NCHUNK - wid + nw_total - 1) // nw_total
    lax.fori_loop(0, nk, do_chunk, 0)


def _segmax(row, x):
    mesh = plsc.VectorSubcoreMesh(core_axis_name="c", subcore_axis_name="s")
    return pl.kernel(
        _segmax_body,
        out_type=jax.ShapeDtypeStruct((_N, _D), jnp.float32),
        mesh=mesh,
        compiler_params=pltpu.CompilerParams(needs_layout_passes=False),
        scratch_types=[
            pltpu.VMEM((_E,), jnp.int32),
            pltpu.VMEM((_LIST,), jnp.int32),
            pltpu.VMEM((_LIST,), jnp.int32),
            pltpu.VMEM((_R, _D), jnp.float32),
            pltpu.VMEM((_G, _D), jnp.float32),
            pltpu.VMEM((_G, _D), jnp.float32),
            pltpu.VMEM((_G, _D), jnp.float32),
            pltpu.VMEM((_G, _D), jnp.float32),
            pltpu.SemaphoreType.DMA,
            pltpu.SemaphoreType.DMA,
            pltpu.SemaphoreType.DMA,
            pltpu.SemaphoreType.DMA,
        ],
    )(row, x)


_BM = 1000


def _mm_body(seg_ref, x_ref, w_ref, b_ref, o_ref):
    seg = seg_ref[...]
    agg = jnp.where(seg == -jnp.inf, x_ref[...], seg)
    acc = lax.dot_general(agg, w_ref[...], (((1,), (1,)), ((), ())),
                          preferred_element_type=jnp.float32)
    o_ref[...] = acc + b_ref[...]


def _matmul(seg, x, W, b2d):
    return pl.pallas_call(
        _mm_body,
        grid=(_N // _BM,),
        in_specs=[
            pl.BlockSpec((_BM, _D), lambda i: (i, 0)),
            pl.BlockSpec((_BM, _D), lambda i: (i, 0)),
            pl.BlockSpec((_D, _D), lambda i: (0, 0)),
            pl.BlockSpec((1, _D), lambda i: (0, 0)),
        ],
        out_specs=pl.BlockSpec((_BM, _D), lambda i: (i, 0)),
        out_shape=jax.ShapeDtypeStruct((_N, _D), jnp.float32),
    )(seg, x, W, b2d)


@jax.jit
def kernel(x, edge_index, W, b):
    row = edge_index[0]
    seg = _segmax(row, x)
    return _matmul(seg, x, W, b.reshape(1, _D))
